# Initial kernel scaffold; baseline (speedup 1.0000x reference)
import jax
import jax.numpy as jnp
from jax import lax
from jax.experimental import pallas as pl
from jax.experimental.pallas import tpu as pltpu
from jax.experimental.pallas import tpu_sc as plsc

N = 16384
D = 64
C = 100000
NC = 2
NS = 16
NW = NC * NS
CPT = C // NW
CPT_PAD = 3136
SLOTS = 512
LCHUNK = 1024
GB = 128
CPROW = 256
LIST_CAP = N + 16


def _body(features, labels, centers, out,
          lab_buf, list_idx, list_crel, counts, slotmap, sums,
          wk_idx, wk_slot, bl_crel, bl_slot, scat_idx, rows,
          cp0, cp1, sem_g, sem_r0, sem_r1, sem_w0, sem_w1):
  wid = lax.axis_index("s") * NC + lax.axis_index("c")
  lo = wid * CPT
  iota16 = lax.iota(jnp.int32, 16)
  zeros16f = jnp.zeros((16,), jnp.float32)
  zeros16i = jnp.zeros((16,), jnp.int32)

  def z_list(i, _):
    list_idx[pl.ds(i * 16, 16)] = zeros16i
    list_crel[pl.ds(i * 16, 16)] = zeros16i
    return 0
  lax.fori_loop(0, LIST_CAP // 16, z_list, 0)
  def z_small(i, _):
    wk_idx[pl.ds(i * 16, 16)] = zeros16i
    wk_slot[pl.ds(i * 16, 16)] = zeros16i
    bl_crel[pl.ds(i * 16, 16)] = zeros16i
    bl_slot[pl.ds(i * 16, 16)] = zeros16i
    return 0
  lax.fori_loop(0, (GB + 32) // 16, z_small, 0)

  def build_chunk(chunk, cur):
    pltpu.sync_copy(labels.at[pl.ds(chunk * LCHUNK, LCHUNK)], lab_buf)
    def group(g, cur):
      lab = lab_buf[pl.ds(g * 16, 16)]
      crel = lab - lo
      pos = iota16 + (chunk * LCHUNK + g * 16)
      m = (crel >= 0) & (crel < CPT)
      plsc.store_compressed(list_idx.at[pl.ds(cur, 16)], pos, mask=m)
      plsc.store_compressed(list_crel.at[pl.ds(cur, 16)], crel, mask=m)
      return cur + jnp.sum(m.astype(jnp.int32))
    return lax.fori_loop(0, LCHUNK // 16, group, cur)
  cur = lax.fori_loop(0, N // LCHUNK, build_chunk, jnp.int32(0))

  def z_counts(z, _):
    counts[pl.ds(z * 16, 16)] = zeros16i
    return 0
  lax.fori_loop(0, CPT_PAD // 16, z_counts, 0)

  def add_count(i, _):
    c = list_crel[i]
    counts[c] = counts[c] + 1
    return 0
  lax.fori_loop(0, cur, add_count, 0)

  def slot_group(z, base):
    cv = counts[pl.ds(z * 16, 16)]
    m = (cv > 0).astype(jnp.int32)
    incl = plsc.cumsum(m)
    slotmap[pl.ds(z * 16, 16)] = base + incl - m
    return base + jnp.sum(m)
  total_touched = lax.fori_loop(0, CPT_PAD // 16, slot_group, jnp.int32(0))

  nfull = CPT // CPROW
  rem = CPT - nfull * CPROW
  chunks = [(i * CPROW, CPROW) for i in range(nfull)]
  if rem:
    chunks.append((nfull * CPROW, rem))
  bufs = [cp0, cp1]
  rsems = [sem_r0, sem_r1]
  wsems = [sem_w0, sem_w1]
  pend_w = [None, None]
  pend_r = [None, None]
  st0, sz0 = chunks[0]
  pend_r[0] = pltpu.async_copy(
      centers.at[pl.ds(lo + st0, sz0)], bufs[0].at[pl.ds(0, sz0)], rsems[0])
  for i, (st, sz) in enumerate(chunks):
    b = i % 2
    pend_r[b].wait()
    if i + 1 < len(chunks):
      nb = (i + 1) % 2
      if pend_w[nb] is not None:
        pend_w[nb].wait()
        pend_w[nb] = None
      nst, nsz = chunks[i + 1]
      pend_r[nb] = pltpu.async_copy(
          centers.at[pl.ds(lo + nst, nsz)], bufs[nb].at[pl.ds(0, nsz)],
          rsems[nb])
    pend_w[b] = pltpu.async_copy(
        bufs[b].at[pl.ds(0, sz)], out.at[pl.ds(lo + st, sz)], wsems[b])
  for b in range(2):
    if pend_w[b] is not None:
      pend_w[b].wait()

  def do_accum_flush(nvalid):
    pltpu.async_copy(features.at[wk_idx.at[pl.ds(0, GB)]], rows, sem_g).wait()
    def acc_row(j, _):
      @pl.when(j < nvalid)
      def _():
        s = wk_slot[j]
        for kk in range(D // 16):
          sv = sums[pl.ds(s * D + kk * 16, 16)]
          rv = rows[j, pl.ds(kk * 16, 16)]
          sums[pl.ds(s * D + kk * 16, 16)] = sv + rv
      return 0
    lax.fori_loop(0, GB, acc_row, 0)

  def do_blend_flush(nvalid):
    for zz in range(GB // 16):
      scat_idx[pl.ds(zz * 16, 16)] = bl_crel[pl.ds(zz * 16, 16)] + lo
    pltpu.async_copy(centers.at[scat_idx], rows, sem_g).wait()
    def blend_row(j, _):
      @pl.when(j < nvalid)
      def _():
        c = bl_crel[j]
        s = bl_slot[j]
        cnt_v = jnp.full((16,), counts[c], jnp.int32).astype(jnp.float32)
        w = 0.5 / cnt_v
        for kk in range(D // 16):
          cvv = rows[j, pl.ds(kk * 16, 16)]
          svv = sums[pl.ds(s * D + kk * 16, 16)]
          rows[j, pl.ds(kk * 16, 16)] = 0.5 * cvv + w * svv
      @pl.when(j >= nvalid)
      def _():
        scat_idx[j] = scat_idx[0]
        for kk in range(D // 16):
          rows[j, pl.ds(kk * 16, 16)] = rows[0, pl.ds(kk * 16, 16)]
      return 0
    lax.fori_loop(0, GB, blend_row, 0)
    pltpu.async_copy(rows, out.at[scat_idx], sem_g).wait()

  def chunk_pass(k, _):
    slot_lo = k * SLOTS
    def z_sums(i, _):
      for kk in range(4):
        sums[pl.ds(i * 64 + kk * 16, 16)] = zeros16f
      return 0
    lax.fori_loop(0, SLOTS * D // 64, z_sums, 0)

    ngroups = (cur + 15) // 16
    def agroup(g, wcur):
      crel16 = list_crel[pl.ds(g * 16, 16)]
      idx16 = list_idx[pl.ds(g * 16, 16)]
      pos = iota16 + g * 16
      slot16 = plsc.load_gather(slotmap, [crel16])
      m = (pos < cur) & (slot16 >= slot_lo) & (slot16 < slot_lo + SLOTS)
      plsc.store_compressed(wk_idx.at[pl.ds(wcur, 16)], idx16, mask=m)
      plsc.store_compressed(wk_slot.at[pl.ds(wcur, 16)], slot16 - slot_lo,
                            mask=m)
      wcur = wcur + jnp.sum(m.astype(jnp.int32))
      @pl.when(wcur >= GB)
      def _():
        do_accum_flush(GB)
        wk_idx[pl.ds(0, 16)] = wk_idx[pl.ds(GB, 16)]
        wk_slot[pl.ds(0, 16)] = wk_slot[pl.ds(GB, 16)]
      return lax.select(wcur >= GB, wcur - GB, wcur)
    wcur = lax.fori_loop(0, ngroups, agroup, jnp.int32(0))
    @pl.when(wcur > 0)
    def _():
      do_accum_flush(wcur)

    def bgroup(z, bcur):
      cv = counts[pl.ds(z * 16, 16)]
      crel16 = iota16 + z * 16
      slot16 = slotmap[pl.ds(z * 16, 16)]
      m = ((cv > 0) & (slot16 >= slot_lo) & (slot16 < slot_lo + SLOTS)
           & (crel16 < CPT))
      plsc.store_compressed(bl_crel.at[pl.ds(bcur, 16)], crel16, mask=m)
      plsc.store_compressed(bl_slot.at[pl.ds(bcur, 16)], slot16 - slot_lo,
                            mask=m)
      bcur = bcur + jnp.sum(m.astype(jnp.int32))
      @pl.when(bcur >= GB)
      def _():
        do_blend_flush(GB)
        bl_crel[pl.ds(0, 16)] = bl_crel[pl.ds(GB, 16)]
        bl_slot[pl.ds(0, 16)] = bl_slot[pl.ds(GB, 16)]
      return lax.select(bcur >= GB, bcur - GB, bcur)
    bcur = lax.fori_loop(0, CPT_PAD // 16, bgroup, jnp.int32(0))
    @pl.when(bcur > 0)
    def _():
      do_blend_flush(bcur)
    return 0

  nchunks = (total_touched + SLOTS - 1) // SLOTS
  lax.fori_loop(0, nchunks, chunk_pass, 0)


@jax.jit
def _run(features, labels, centers):
  mesh = plsc.VectorSubcoreMesh(core_axis_name="c", subcore_axis_name="s",
                                num_cores=NC, num_subcores=NS)
  kern = pl.kernel(
      _body,
      out_type=jax.ShapeDtypeStruct((C, D), jnp.float32),
      mesh=mesh,
      scratch_types=[
          pltpu.VMEM((LCHUNK,), jnp.int32),
          pltpu.VMEM((LIST_CAP,), jnp.int32),
          pltpu.VMEM((LIST_CAP,), jnp.int32),
          pltpu.VMEM((CPT_PAD,), jnp.int32),
          pltpu.VMEM((CPT_PAD,), jnp.int32),
          pltpu.VMEM((SLOTS * D,), jnp.float32),
          pltpu.VMEM((GB + 32,), jnp.int32),
          pltpu.VMEM((GB + 32,), jnp.int32),
          pltpu.VMEM((GB + 32,), jnp.int32),
          pltpu.VMEM((GB + 32,), jnp.int32),
          pltpu.VMEM((GB,), jnp.int32),
          pltpu.VMEM((GB, D), jnp.float32),
          pltpu.VMEM((CPROW, D), jnp.float32),
          pltpu.VMEM((CPROW, D), jnp.float32),
          pltpu.SemaphoreType.DMA,
          pltpu.SemaphoreType.DMA,
          pltpu.SemaphoreType.DMA,
          pltpu.SemaphoreType.DMA,
          pltpu.SemaphoreType.DMA,
      ],
  )
  return kern(features, labels, centers)


def kernel(features, labels, centers):
  return _run(features, labels.astype(jnp.int32), centers)


# speedup vs baseline: 1.1295x; 1.1295x over previous
import jax
import jax.numpy as jnp
from jax import lax
from jax.experimental import pallas as pl
from jax.experimental.pallas import tpu as pltpu
from jax.experimental.pallas import tpu_sc as plsc

N = 16384
D = 64
C = 100000
NC = 2
NS = 16
NW = NC * NS
CPT = 3128
CPT_LAST = C - (NW - 1) * CPT
CPT_PAD = 3136
SLOTS = 512
LCHUNK = 1024
GB = 128
CPROW = 256
LIST_CAP = N + 16


def _body(features, labels, centers, out,
          lab_buf, list_idx, list_crel, counts, slotmap, sums,
          wk_idx, wk_slot, bl_crel, bl_slot, scat_idx, rows,
          cp0, cp1, sem_g, sem_r0, sem_r1, sem_w0, sem_w1):
  wid = lax.axis_index("s") * NC + lax.axis_index("c")
  lo = wid * CPT
  iota16 = lax.iota(jnp.int32, 16)
  zeros16f = jnp.zeros((16,), jnp.float32)
  zeros16i = jnp.zeros((16,), jnp.int32)

  def compress_store(ref, x, m, base):
    mi = m.astype(jnp.int32)
    dest = base + plsc.cumsum(mi) - mi
    plsc.store_scatter(ref, [dest], x, mask=m)
    return plsc.all_reduce_population_count(m)[0]

  def z_list(i, _):
    list_idx[pl.ds(i * 16, 16)] = zeros16i
    list_crel[pl.ds(i * 16, 16)] = zeros16i
    return 0
  lax.fori_loop(0, LIST_CAP // 16, z_list, 0)
  def z_small(i, _):
    wk_idx[pl.ds(i * 16, 16)] = zeros16i
    wk_slot[pl.ds(i * 16, 16)] = zeros16i
    bl_crel[pl.ds(i * 16, 16)] = zeros16i
    bl_slot[pl.ds(i * 16, 16)] = zeros16i
    return 0
  lax.fori_loop(0, (GB + 32) // 16, z_small, 0)

  def build_chunk(chunk, cur):
    pltpu.sync_copy(labels.at[pl.ds(chunk * LCHUNK, LCHUNK)], lab_buf)
    def group(g, cur):
      lab = lab_buf[pl.ds(g * 16, 16)]
      crel = lab - lo
      pos = iota16 + (chunk * LCHUNK + g * 16)
      m = (crel >= 0) & (crel < CPT)
      compress_store(list_idx, pos, m, cur)
      return cur + compress_store(list_crel, crel, m, cur)
    return lax.fori_loop(0, LCHUNK // 16, group, cur)
  cur = lax.fori_loop(0, N // LCHUNK, build_chunk, jnp.int32(0))

  def z_counts(z, _):
    counts[pl.ds(z * 16, 16)] = zeros16i
    return 0
  lax.fori_loop(0, CPT_PAD // 16, z_counts, 0)

  one_hot0 = (iota16 == 0).astype(jnp.int32)
  def add_count(i, _):
    c = list_crel[pl.ds(i, 16)][0]
    cnts = counts[pl.ds(c, 16)]
    counts[pl.ds(c, 16)] = cnts + one_hot0
    return 0
  lax.fori_loop(0, cur, add_count, 0)

  def slot_group(z, base):
    cv = counts[pl.ds(z * 16, 16)]
    mb = cv > 0
    m = mb.astype(jnp.int32)
    incl = plsc.cumsum(m)
    slotmap[pl.ds(z * 16, 16)] = base + incl - m
    return base + plsc.all_reduce_population_count(mb)[0]
  total_touched = lax.fori_loop(0, CPT_PAD // 16, slot_group, jnp.int32(0))

  bufs = [cp0, cp1]
  rsems = [sem_r0, sem_r1]
  wsems = [sem_w0, sem_w1]

  def copy_range(nrows):
    nfull = nrows // CPROW
    rem = nrows - nfull * CPROW
    chunks = [(i * CPROW, CPROW) for i in range(nfull)]
    if rem:
      chunks.append((nfull * CPROW, rem))
    pend_w = [None, None]
    pend_r = [None, None]
    st0, sz0 = chunks[0]
    pend_r[0] = pltpu.async_copy(
        centers.at[pl.ds(lo + st0, sz0)], bufs[0].at[pl.ds(0, sz0)], rsems[0])
    for i, (st, sz) in enumerate(chunks):
      b = i % 2
      pend_r[b].wait()
      if i + 1 < len(chunks):
        nb = (i + 1) % 2
        if pend_w[nb] is not None:
          pend_w[nb].wait()
          pend_w[nb] = None
        nst, nsz = chunks[i + 1]
        pend_r[nb] = pltpu.async_copy(
            centers.at[pl.ds(lo + nst, nsz)], bufs[nb].at[pl.ds(0, nsz)],
            rsems[nb])
      pend_w[b] = pltpu.async_copy(
          bufs[b].at[pl.ds(0, sz)], out.at[pl.ds(lo + st, sz)], wsems[b])
    for b in range(2):
      if pend_w[b] is not None:
        pend_w[b].wait()

  @pl.when(wid < NW - 1)
  def _():
    copy_range(CPT)

  @pl.when(wid == NW - 1)
  def _():
    copy_range(CPT_LAST)

  def do_accum_flush(nvalid):
    pltpu.async_copy(features.at[wk_idx.at[pl.ds(0, GB)]], rows, sem_g).wait()
    def acc_row(j, _):
      @pl.when(j < nvalid)
      def _():
        s = wk_slot[pl.ds(j, 16)][0]
        for kk in range(D // 16):
          sv = sums[pl.ds(s * D + kk * 16, 16)]
          rv = rows[j, pl.ds(kk * 16, 16)]
          sums[pl.ds(s * D + kk * 16, 16)] = sv + rv
      return 0
    lax.fori_loop(0, GB, acc_row, 0)

  def do_blend_flush(nvalid):
    for zz in range(GB // 16):
      scat_idx[pl.ds(zz * 16, 16)] = bl_crel[pl.ds(zz * 16, 16)] + lo
    pltpu.async_copy(centers.at[scat_idx], rows, sem_g).wait()
    def blend_row(j, _):
      @pl.when(j < nvalid)
      def _():
        c = bl_crel[pl.ds(j, 16)][0]
        s = bl_slot[pl.ds(j, 16)][0]
        cnt_v = jnp.full((16,), counts[pl.ds(c, 16)][0],
                         jnp.int32).astype(jnp.float32)
        w = 0.5 / cnt_v
        for kk in range(D // 16):
          cvv = rows[j, pl.ds(kk * 16, 16)]
          svv = sums[pl.ds(s * D + kk * 16, 16)]
          rows[j, pl.ds(kk * 16, 16)] = 0.5 * cvv + w * svv
      return 0
    lax.fori_loop(0, GB, blend_row, 0)
    s0 = scat_idx[pl.ds(0, 16)][0]
    for zz in range(GB // 16):
      lane_pos = iota16 + zz * 16
      curv = scat_idx[pl.ds(zz * 16, 16)]
      scat_idx[pl.ds(zz * 16, 16)] = jnp.where(lane_pos >= nvalid, s0, curv)
    row0s = [rows[0, pl.ds(kk * 16, 16)] for kk in range(D // 16)]
    def tail_row(j, _):
      for kk in range(D // 16):
        rows[j, pl.ds(kk * 16, 16)] = row0s[kk]
      return 0
    lax.fori_loop(nvalid, GB, tail_row, 0)
    pltpu.async_copy(rows, out.at[scat_idx], sem_g).wait()

  def chunk_pass(k, _):
    slot_lo = k * SLOTS
    def z_sums(i, _):
      for kk in range(4):
        sums[pl.ds(i * 64 + kk * 16, 16)] = zeros16f
      return 0
    lax.fori_loop(0, SLOTS * D // 64, z_sums, 0)

    ngroups = (cur + 15) // 16
    def agroup(g, wcur):
      crel16 = list_crel[pl.ds(g * 16, 16)]
      idx16 = list_idx[pl.ds(g * 16, 16)]
      pos = iota16 + g * 16
      slot16 = plsc.load_gather(slotmap, [crel16])
      m = (pos < cur) & (slot16 >= slot_lo) & (slot16 < slot_lo + SLOTS)
      compress_store(wk_idx, idx16, m, wcur)
      wcur = wcur + compress_store(wk_slot, slot16 - slot_lo, m, wcur)
      @pl.when(wcur >= GB)
      def _():
        do_accum_flush(GB)
        wk_idx[pl.ds(0, 16)] = wk_idx[pl.ds(GB, 16)]
        wk_slot[pl.ds(0, 16)] = wk_slot[pl.ds(GB, 16)]
      return lax.select(wcur >= GB, wcur - GB, wcur)
    wcur = lax.fori_loop(0, ngroups, agroup, jnp.int32(0))
    @pl.when(wcur > 0)
    def _():
      do_accum_flush(wcur)

    def bgroup(z, bcur):
      cv = counts[pl.ds(z * 16, 16)]
      crel16 = iota16 + z * 16
      slot16 = slotmap[pl.ds(z * 16, 16)]
      m = ((cv > 0) & (slot16 >= slot_lo) & (slot16 < slot_lo + SLOTS)
           & (crel16 < CPT))
      compress_store(bl_crel, crel16, m, bcur)
      bcur = bcur + compress_store(bl_slot, slot16 - slot_lo, m, bcur)
      @pl.when(bcur >= GB)
      def _():
        do_blend_flush(GB)
        bl_crel[pl.ds(0, 16)] = bl_crel[pl.ds(GB, 16)]
        bl_slot[pl.ds(0, 16)] = bl_slot[pl.ds(GB, 16)]
      return lax.select(bcur >= GB, bcur - GB, bcur)
    bcur = lax.fori_loop(0, CPT_PAD // 16, bgroup, jnp.int32(0))
    @pl.when(bcur > 0)
    def _():
      do_blend_flush(bcur)
    return 0

  nchunks = (total_touched + SLOTS - 1) // SLOTS
  lax.fori_loop(0, nchunks, chunk_pass, 0)


@jax.jit
def _run(features, labels, centers):
  mesh = plsc.VectorSubcoreMesh(core_axis_name="c", subcore_axis_name="s",
                                num_cores=NC, num_subcores=NS)
  kern = pl.kernel(
      _body,
      out_type=jax.ShapeDtypeStruct((C, D), jnp.float32),
      mesh=mesh,
      compiler_params=pltpu.CompilerParams(needs_layout_passes=False,
                                           use_tc_tiling_on_sc=False),
      scratch_types=[
          pltpu.VMEM((LCHUNK,), jnp.int32),
          pltpu.VMEM((LIST_CAP,), jnp.int32),
          pltpu.VMEM((LIST_CAP,), jnp.int32),
          pltpu.VMEM((CPT_PAD + 16,), jnp.int32),
          pltpu.VMEM((CPT_PAD,), jnp.int32),
          pltpu.VMEM((SLOTS * D,), jnp.float32),
          pltpu.VMEM((GB + 32,), jnp.int32),
          pltpu.VMEM((GB + 32,), jnp.int32),
          pltpu.VMEM((GB + 32,), jnp.int32),
          pltpu.VMEM((GB + 32,), jnp.int32),
          pltpu.VMEM((GB,), jnp.int32),
          pltpu.VMEM((GB, D), jnp.float32),
          pltpu.VMEM((CPROW, D), jnp.float32),
          pltpu.VMEM((CPROW, D), jnp.float32),
          pltpu.SemaphoreType.DMA,
          pltpu.SemaphoreType.DMA,
          pltpu.SemaphoreType.DMA,
          pltpu.SemaphoreType.DMA,
          pltpu.SemaphoreType.DMA,
      ],
  )
  return kern(features, labels, centers)


def kernel(features, labels, centers):
  return _run(features, labels.astype(jnp.int32), centers)
